# baseline (speedup 1.0000x reference)
import functools

import jax
import jax.numpy as jnp
from jax import lax
from jax.experimental import pallas as pl
from jax.experimental.pallas import tpu as pltpu
from jax.experimental.pallas import tpu_sc as plsc

NUM_CLASSES = 100
LAMBDA_ = 3.0
BETA = 0.7


def _make_sc_gather(batch, ncls):
    info = plsc.get_sparse_core_info()
    nc, ns = info.num_cores, info.num_subcores
    nw = nc * ns
    assert batch % (8 * nw) == 0
    b_per_w = batch // nw
    mesh = plsc.VectorSubcoreMesh(core_axis_name="c", subcore_axis_name="s")

    @functools.partial(
        pl.kernel,
        mesh=mesh,
        out_type=jax.ShapeDtypeStruct((batch, ncls), jnp.float32),
        scratch_types=[
            pltpu.VMEM((b_per_w,), jnp.int32),
            pltpu.VMEM((b_per_w, ncls), jnp.float32),
            pltpu.SemaphoreType.DMA,
        ],
        compiler_params=pltpu.CompilerParams(use_tc_tiling_on_sc=False),
    )
    def gather_kernel(idx_hbm, table_hbm, out_hbm, idx_v, rows_v, sem):
        wid = lax.axis_index("s") * nc + lax.axis_index("c")
        base = wid * b_per_w
        pltpu.sync_copy(idx_hbm.at[pl.ds(base, b_per_w)], idx_v)
        pltpu.async_copy(table_hbm.at[idx_v], rows_v, sem).wait()
        pltpu.sync_copy(rows_v, out_hbm.at[pl.ds(base, b_per_w)])

    return gather_kernel


def _loss_body(out_ref, lab_ref, gath_ref, res_ref):
    x = out_ref[...]
    b, c = x.shape
    m = jnp.max(x, axis=1, keepdims=True)
    e = jnp.exp(x - m)
    s = jnp.sum(e, axis=1, keepdims=True)
    y = jnp.clip(e / s, 0.0001, 1.0 - 0.0001)
    y_norm = y / jnp.sum(y, axis=1, keepdims=True)
    t_new = BETA * gath_ref[...] + (1.0 - BETA) * y_norm
    logp = (x - m) - jnp.log(s)
    lab = lab_ref[...]
    cls = lax.broadcasted_iota(jnp.int32, (b, c), 1)
    ce_per = -jnp.sum(jnp.where(cls == lab, logp, 0.0), axis=1)
    elr_per = jnp.log(1.0 - jnp.sum(t_new * y, axis=1))
    res_ref[0, 0] = (jnp.sum(ce_per) + LAMBDA_ * jnp.sum(elr_per)) / b


def kernel(index, output, label, target):
    batch, ncls = output.shape
    idx = index.astype(jnp.int32)
    gathered = _make_sc_gather(batch, ncls)(idx, target)
    res = pl.pallas_call(
        _loss_body,
        out_shape=jax.ShapeDtypeStruct((1, 1), jnp.float32),
        in_specs=[
            pl.BlockSpec(memory_space=pltpu.VMEM),
            pl.BlockSpec(memory_space=pltpu.VMEM),
            pl.BlockSpec(memory_space=pltpu.VMEM),
        ],
        out_specs=pl.BlockSpec(memory_space=pltpu.SMEM),
    )(output, label.astype(jnp.int32).reshape(batch, 1), gathered)
    return res[0, 0]


# speedup vs baseline: 5.8337x; 5.8337x over previous
import functools

import jax
import jax.numpy as jnp
from jax import lax
from jax.experimental import pallas as pl
from jax.experimental.pallas import tpu as pltpu
from jax.experimental.pallas import tpu_sc as plsc

NUM_CLASSES = 100
LAMBDA_ = 3.0
BETA = 0.7
NBUF = 8
LANES = 16
CPAD = 112


def _make_sc_gather(batch, ncls):
    info = plsc.get_sparse_core_info()
    nc, ns = info.num_cores, info.num_subcores
    nw = nc * ns
    assert batch % (8 * nw) == 0
    b_per_w = batch // nw
    nchunk = CPAD // LANES
    mesh = plsc.VectorSubcoreMesh(core_axis_name="c", subcore_axis_name="s")

    nchunk_idx = b_per_w // LANES

    @functools.partial(
        pl.kernel,
        mesh=mesh,
        out_type=jax.ShapeDtypeStruct((batch, CPAD), jnp.float32),
        scratch_types=[
            pltpu.VMEM((b_per_w,), jnp.int32),
            pltpu.VMEM((2 * LANES, 8, ncls), jnp.float32),
            pltpu.VMEM((b_per_w, CPAD), jnp.float32),
            pltpu.SemaphoreType.DMA,
        ],
        compiler_params=pltpu.CompilerParams(needs_layout_passes=False),
    )
    def gather_kernel(
        idx_hbm, table_hbm, out_hbm, idx_v, tiles_v, rows_v, sem
    ):
        wid = lax.axis_index("s") * nc + lax.axis_index("c")
        base = wid * b_per_w
        pltpu.sync_copy(idx_hbm.at[pl.ds(base, b_per_w)], idx_v)
        lane = lax.iota(jnp.int32, LANES)

        def issue_chunk(ci, slot_base):
            vec = idx_v[pl.ds(ci * LANES, LANES)]
            for k in range(LANES):
                row0 = (vec[k] // 8) * 8
                pltpu.async_copy(
                    table_hbm.at[pl.ds(row0, 8), :],
                    tiles_v.at[slot_base + k],
                    sem,
                )

        def drain_chunk():
            for _ in range(LANES):
                pltpu.make_async_copy(
                    table_hbm.at[pl.ds(0, 8), :], tiles_v.at[0], sem
                ).wait()

        def extract_chunk(ci, slot_base):
            vec = idx_v[pl.ds(ci * LANES, LANES)]
            for k in range(LANES):
                i = ci * LANES + k
                slot = jnp.full((LANES,), slot_base + k, jnp.int32)
                sub = jnp.full((LANES,), vec[k] % 8, jnp.int32)
                for j in range(nchunk):
                    col = jnp.minimum(lane + (j * LANES), ncls - 1)
                    v = plsc.load_gather(tiles_v, [slot, sub, col])
                    rows_v[i, pl.ds(j * LANES, LANES)] = v

        issue_chunk(0, 0)

        def body(g, carry):
            cur_base = (g % 2) * LANES
            drain_chunk()

            @pl.when(g + 1 < nchunk_idx)
            def _():
                issue_chunk(g + 1, (g + 1) % 2 * LANES)

            extract_chunk(g, cur_base)
            return carry

        lax.fori_loop(0, nchunk_idx, body, 0)
        pltpu.sync_copy(rows_v, out_hbm.at[pl.ds(base, b_per_w)])

    return gather_kernel


def _loss_body(out_ref, lab_ref, gath_ref, res_ref):
    x = out_ref[...]
    b, c = x.shape
    m = jnp.max(x, axis=1, keepdims=True)
    e = jnp.exp(x - m)
    s = jnp.sum(e, axis=1, keepdims=True)
    y = jnp.clip(e / s, 0.0001, 1.0 - 0.0001)
    y_norm = y / jnp.sum(y, axis=1, keepdims=True)
    t_new = BETA * gath_ref[:, :c] + (1.0 - BETA) * y_norm
    logp = (x - m) - jnp.log(s)
    lab = lab_ref[...]
    cls = lax.broadcasted_iota(jnp.int32, (b, c), 1)
    ce_per = -jnp.sum(jnp.where(cls == lab, logp, 0.0), axis=1)
    elr_per = jnp.log(1.0 - jnp.sum(t_new * y, axis=1))
    res_ref[0, 0] = (jnp.sum(ce_per) + LAMBDA_ * jnp.sum(elr_per)) / b


def kernel(index, output, label, target):
    batch, ncls = output.shape
    idx = index.astype(jnp.int32)
    gathered = _make_sc_gather(batch, ncls)(idx, target)
    res = pl.pallas_call(
        _loss_body,
        out_shape=jax.ShapeDtypeStruct((1, 1), jnp.float32),
        in_specs=[
            pl.BlockSpec(memory_space=pltpu.VMEM),
            pl.BlockSpec(memory_space=pltpu.VMEM),
            pl.BlockSpec(memory_space=pltpu.VMEM),
        ],
        out_specs=pl.BlockSpec(memory_space=pltpu.SMEM),
    )(output, label.astype(jnp.int32).reshape(batch, 1), gathered)
    return res[0, 0]


# speedup vs baseline: 22.0809x; 3.7850x over previous
import functools

import jax
import jax.numpy as jnp
from jax import lax
from jax.experimental import pallas as pl
from jax.experimental.pallas import tpu as pltpu
from jax.experimental.pallas import tpu_sc as plsc

NUM_CLASSES = 100
LAMBDA_ = 3.0
BETA = 0.7
LANES = 16
CPAD = 112
NSLOT = 4


def _make_sc_gather(batch, ncls, nexamp):
    info = plsc.get_sparse_core_info()
    nc, ns = info.num_cores, info.num_subcores
    nw = nc * ns
    assert batch % (LANES * nw) == 0
    b_per_w = batch // nw
    nchunk_c = CPAD // LANES
    nchunk_i = b_per_w // LANES
    max_col0 = nexamp - 128
    mesh = plsc.VectorSubcoreMesh(core_axis_name="c", subcore_axis_name="s")

    @functools.partial(
        pl.kernel,
        mesh=mesh,
        out_type=jax.ShapeDtypeStruct((batch, CPAD), jnp.float32),
        scratch_types=[
            pltpu.VMEM((b_per_w,), jnp.int32),
        ]
        + [pltpu.VMEM((ncls, 128), jnp.float32)] * NSLOT
        + [
            pltpu.VMEM((b_per_w, CPAD), jnp.float32),
        ]
        + [pltpu.SemaphoreType.DMA] * NSLOT,
        compiler_params=pltpu.CompilerParams(needs_layout_passes=False),
    )
    def gather_kernel(idx_hbm, table_hbm, out_hbm, idx_v, *rest):
        tiles = rest[:NSLOT]
        rows_v = rest[NSLOT]
        sems = rest[NSLOT + 1 :]
        wid = lax.axis_index("s") * nc + lax.axis_index("c")
        base = wid * b_per_w
        pltpu.sync_copy(idx_hbm.at[pl.ds(base, b_per_w)], idx_v)
        lane = lax.iota(jnp.int32, LANES)

        def col0_of(v):
            return pl.multiple_of((v // 128) * 128, 128)

        def issue(v, k):
            pltpu.async_copy(
                table_hbm.at[:, pl.ds(col0_of(v), 128)],
                tiles[k % NSLOT],
                sems[k % NSLOT],
            )

        def wait(k):
            pltpu.make_async_copy(
                table_hbm.at[:, pl.ds(0, 128)],
                tiles[k % NSLOT],
                sems[k % NSLOT],
            ).wait()

        def extract(v, k, i):
            col = jnp.full((LANES,), v - col0_of(v), jnp.int32)
            for j in range(nchunk_c):
                row = jnp.minimum(lane + (j * LANES), ncls - 1)
                g = plsc.load_gather(tiles[k % NSLOT], [row, col])
                rows_v[i, pl.ds(j * LANES, LANES)] = g

        def body(g, carry):
            vec = idx_v[pl.ds(g * LANES, LANES)]
            for k in range(NSLOT - 1):
                issue(vec[k], k)
            for k in range(LANES):
                if k + NSLOT - 1 < LANES:
                    issue(vec[k + NSLOT - 1], k + NSLOT - 1)
                wait(k)
                extract(vec[k], k, g * LANES + k)
            return carry

        lax.fori_loop(0, nchunk_i, body, 0)
        pltpu.sync_copy(rows_v, out_hbm.at[pl.ds(base, b_per_w)])

    return gather_kernel


def _loss_body(out_ref, lab_ref, gath_ref, res_ref):
    x = out_ref[...]
    b, c = x.shape
    m = jnp.max(x, axis=1, keepdims=True)
    e = jnp.exp(x - m)
    s = jnp.sum(e, axis=1, keepdims=True)
    y = jnp.clip(e / s, 0.0001, 1.0 - 0.0001)
    y_norm = y / jnp.sum(y, axis=1, keepdims=True)
    t_new = BETA * gath_ref[:, :c] + (1.0 - BETA) * y_norm
    logp = (x - m) - jnp.log(s)
    lab = lab_ref[...]
    cls = lax.broadcasted_iota(jnp.int32, (b, c), 1)
    ce_per = -jnp.sum(jnp.where(cls == lab, logp, 0.0), axis=1)
    elr_per = jnp.log(1.0 - jnp.sum(t_new * y, axis=1))
    res_ref[0, 0] = (jnp.sum(ce_per) + LAMBDA_ * jnp.sum(elr_per)) / b


def kernel(index, output, label, target):
    batch, ncls = output.shape
    nexamp = target.shape[0]
    idx = index.astype(jnp.int32)
    gathered = _make_sc_gather(batch, ncls, nexamp)(idx, target.T)
    res = pl.pallas_call(
        _loss_body,
        out_shape=jax.ShapeDtypeStruct((1, 1), jnp.float32),
        in_specs=[
            pl.BlockSpec(memory_space=pltpu.VMEM),
            pl.BlockSpec(memory_space=pltpu.VMEM),
            pl.BlockSpec(memory_space=pltpu.VMEM),
        ],
        out_specs=pl.BlockSpec(memory_space=pltpu.SMEM),
    )(output, label.astype(jnp.int32).reshape(batch, 1), gathered)
    return res[0, 0]


# speedup vs baseline: 22.3260x; 1.0111x over previous
import functools

import jax
import jax.numpy as jnp
from jax import lax
from jax.experimental import pallas as pl
from jax.experimental.pallas import tpu as pltpu
from jax.experimental.pallas import tpu_sc as plsc

NUM_CLASSES = 100
LAMBDA_ = 3.0
BETA = 0.7
LANES = 16
CPAD = 112
NSLOT = 4


def _make_sc_gather(batch, ncls):
    info = plsc.get_sparse_core_info()
    nc, ns = info.num_cores, info.num_subcores
    nw = nc * ns
    assert batch % (LANES * nw) == 0 and LANES % NSLOT == 0
    b_per_w = batch // nw
    nchunk_c = CPAD // LANES
    nchunk_i = b_per_w // LANES
    mesh = plsc.VectorSubcoreMesh(core_axis_name="c", subcore_axis_name="s")

    @functools.partial(
        pl.kernel,
        mesh=mesh,
        out_type=jax.ShapeDtypeStruct((CPAD, batch), jnp.float32),
        scratch_types=[
            pltpu.VMEM((b_per_w,), jnp.int32),
        ]
        + [pltpu.VMEM((ncls, 128), jnp.float32)] * NSLOT
        + [
            pltpu.VMEM((CPAD, b_per_w), jnp.float32),
        ]
        + [pltpu.SemaphoreType.DMA] * NSLOT,
        compiler_params=pltpu.CompilerParams(needs_layout_passes=False),
    )
    def gather_kernel(idx_hbm, table_hbm, out_hbm, idx_v, *rest):
        tiles = rest[:NSLOT]
        rows_t = rest[NSLOT]
        sems = rest[NSLOT + 1 :]
        wid = lax.axis_index("s") * nc + lax.axis_index("c")
        base = wid * b_per_w
        pltpu.sync_copy(idx_hbm.at[pl.ds(base, b_per_w)], idx_v)
        lane = lax.iota(jnp.int32, LANES)

        def col0_of(v):
            return pl.multiple_of((v // 128) * 128, 128)

        def issue(v, k):
            pltpu.async_copy(
                table_hbm.at[:, pl.ds(col0_of(v), 128)],
                tiles[k % NSLOT],
                sems[k % NSLOT],
            )

        def wait(k):
            pltpu.make_async_copy(
                table_hbm.at[:, pl.ds(0, 128)],
                tiles[k % NSLOT],
                sems[k % NSLOT],
            ).wait()

        def extract(v, k, i_local):
            col = jnp.full((LANES,), v - col0_of(v), jnp.int32)
            out_col = jnp.full((LANES,), i_local, jnp.int32)
            for j in range(nchunk_c):
                row = jnp.minimum(lane + (j * LANES), ncls - 1)
                g = plsc.load_gather(tiles[k % NSLOT], [row, col])
                plsc.store_scatter(rows_t, [lane + (j * LANES), out_col], g)

        vec0 = idx_v[pl.ds(0, LANES)]
        for k in range(NSLOT - 1):
            issue(vec0[k], k)

        def body(g, carry):
            vec = idx_v[pl.ds(g * LANES, LANES)]
            vec_next = idx_v[
                pl.ds(jnp.minimum(g + 1, nchunk_i - 1) * LANES, LANES)
            ]
            for k in range(LANES):
                wait(k)
                extract(vec[k], k, g * LANES + k)
                nxt = k + NSLOT - 1
                if nxt < LANES:
                    issue(vec[nxt], nxt)
                else:

                    @pl.when(g + 1 < nchunk_i)
                    def _():
                        issue(vec_next[nxt - LANES], nxt)

            return carry

        lax.fori_loop(0, nchunk_i, body, 0)
        pltpu.sync_copy(rows_t, out_hbm.at[:, pl.ds(base, b_per_w)])

    return gather_kernel


def _stage1_body(xt_ref, lab_ref, y_ref, aux_ref, ce_ref):
    x = xt_ref[...]
    c, b = x.shape
    m = jnp.max(x, axis=0, keepdims=True)
    e = jnp.exp(x - m)
    s = jnp.sum(e, axis=0, keepdims=True)
    y = jnp.clip(e / s, 0.0001, 1.0 - 0.0001)
    y_norm = y / jnp.sum(y, axis=0, keepdims=True)
    y_ref[...] = y
    aux_ref[0:1, :] = (1.0 - BETA) * jnp.sum(y_norm * y, axis=0, keepdims=True)
    logp = (x - m) - jnp.log(s)
    cls = lax.broadcasted_iota(jnp.int32, (c, b), 0)
    hit = cls == lab_ref[...]
    ce_ref[0, 0] = -jnp.sum(jnp.where(hit, logp, 0.0)) / b


def _stage2_body(gt_ref, y_ref, aux_ref, ce_ref, res_ref):
    c, b = y_ref.shape
    g = gt_ref[0:c, :]
    y = y_ref[...]
    dot = BETA * jnp.sum(g * y, axis=0, keepdims=True) + aux_ref[0:1, :]
    elr = jnp.log(1.0 - dot)
    res_ref[0, 0] = ce_ref[0, 0] + LAMBDA_ * (jnp.sum(elr) / b)


def kernel(index, output, label, target):
    batch, ncls = output.shape
    idx = index.astype(jnp.int32)
    y_t, aux, ce = pl.pallas_call(
        _stage1_body,
        out_shape=(
            jax.ShapeDtypeStruct((ncls, batch), jnp.float32),
            jax.ShapeDtypeStruct((8, batch), jnp.float32),
            jax.ShapeDtypeStruct((1, 1), jnp.float32),
        ),
        in_specs=[
            pl.BlockSpec(memory_space=pltpu.VMEM),
            pl.BlockSpec(memory_space=pltpu.VMEM),
        ],
        out_specs=(
            pl.BlockSpec(memory_space=pltpu.VMEM),
            pl.BlockSpec(memory_space=pltpu.VMEM),
            pl.BlockSpec(memory_space=pltpu.SMEM),
        ),
    )(output.T, label.astype(jnp.int32).reshape(1, batch))
    gathered_t = _make_sc_gather(batch, ncls)(idx, target.T)
    res = pl.pallas_call(
        _stage2_body,
        out_shape=jax.ShapeDtypeStruct((1, 1), jnp.float32),
        in_specs=[
            pl.BlockSpec(memory_space=pltpu.VMEM),
            pl.BlockSpec(memory_space=pltpu.VMEM),
            pl.BlockSpec(memory_space=pltpu.VMEM),
            pl.BlockSpec(memory_space=pltpu.SMEM),
        ],
        out_specs=pl.BlockSpec(memory_space=pltpu.SMEM),
    )(gathered_t, y_t, aux, ce)
    return res[0, 0]


# speedup vs baseline: 26.3635x; 1.1808x over previous
import functools

import jax
import jax.numpy as jnp
from jax import lax
from jax.experimental import pallas as pl
from jax.experimental.pallas import tpu as pltpu
from jax.experimental.pallas import tpu_sc as plsc

NUM_CLASSES = 100
LAMBDA_ = 3.0
BETA = 0.7
LANES = 16
CPAD = 112
NSLOT = 8


def _make_sc_gather(batch, ncls):
    info = plsc.get_sparse_core_info()
    nc, ns = info.num_cores, info.num_subcores
    nw = nc * ns
    assert batch % (LANES * nw) == 0 and LANES % NSLOT == 0
    b_per_w = batch // nw
    nchunk_c = CPAD // LANES
    nchunk_i = b_per_w // LANES
    mesh = plsc.VectorSubcoreMesh(core_axis_name="c", subcore_axis_name="s")

    @functools.partial(
        pl.kernel,
        mesh=mesh,
        out_type=jax.ShapeDtypeStruct((CPAD, batch), jnp.float32),
        scratch_types=[
            pltpu.VMEM((b_per_w,), jnp.int32),
        ]
        + [pltpu.VMEM((ncls, 128), jnp.float32)] * NSLOT
        + [
            pltpu.VMEM((CPAD, b_per_w), jnp.float32),
        ]
        + [pltpu.SemaphoreType.DMA] * NSLOT,
        compiler_params=pltpu.CompilerParams(needs_layout_passes=False),
    )
    def gather_kernel(idx_hbm, table_hbm, out_hbm, idx_v, *rest):
        tiles = rest[:NSLOT]
        rows_t = rest[NSLOT]
        sems = rest[NSLOT + 1 :]
        wid = lax.axis_index("s") * nc + lax.axis_index("c")
        base = wid * b_per_w
        pltpu.sync_copy(idx_hbm.at[pl.ds(base, b_per_w)], idx_v)
        lane = lax.iota(jnp.int32, LANES)

        def col0_of(v):
            return pl.multiple_of((v // 128) * 128, 128)

        def issue(v, k):
            pltpu.async_copy(
                table_hbm.at[:, pl.ds(col0_of(v), 128)],
                tiles[k % NSLOT],
                sems[k % NSLOT],
            )

        def wait(k):
            pltpu.make_async_copy(
                table_hbm.at[:, pl.ds(0, 128)],
                tiles[k % NSLOT],
                sems[k % NSLOT],
            ).wait()

        def extract(v, k, i_local):
            col = jnp.full((LANES,), v - col0_of(v), jnp.int32)
            out_col = jnp.full((LANES,), i_local, jnp.int32)
            for j in range(nchunk_c):
                row = jnp.minimum(lane + (j * LANES), ncls - 1)
                g = plsc.load_gather(tiles[k % NSLOT], [row, col])
                plsc.store_scatter(rows_t, [lane + (j * LANES), out_col], g)

        vec0 = idx_v[pl.ds(0, LANES)]
        for k in range(NSLOT - 1):
            issue(vec0[k], k)

        def body(g, carry):
            vec = idx_v[pl.ds(g * LANES, LANES)]
            vec_next = idx_v[
                pl.ds(jnp.minimum(g + 1, nchunk_i - 1) * LANES, LANES)
            ]
            for k in range(LANES):
                wait(k)
                extract(vec[k], k, g * LANES + k)
                nxt = k + NSLOT - 1
                if nxt < LANES:
                    issue(vec[nxt], nxt)
                else:

                    @pl.when(g + 1 < nchunk_i)
                    def _():
                        issue(vec_next[nxt - LANES], nxt)

            return carry

        lax.fori_loop(0, nchunk_i, body, 0)
        pltpu.sync_copy(rows_t, out_hbm.at[:, pl.ds(base, b_per_w)])

    return gather_kernel


def _stage1_body(xt_ref, lab_ref, y_ref, aux_ref, ce_ref):
    x = xt_ref[...]
    c, b = x.shape
    m = jnp.max(x, axis=0, keepdims=True)
    e = jnp.exp(x - m)
    s = jnp.sum(e, axis=0, keepdims=True)
    y = jnp.clip(e / s, 0.0001, 1.0 - 0.0001)
    y_norm = y / jnp.sum(y, axis=0, keepdims=True)
    y_ref[...] = y
    aux_ref[0:1, :] = (1.0 - BETA) * jnp.sum(y_norm * y, axis=0, keepdims=True)
    logp = (x - m) - jnp.log(s)
    cls = lax.broadcasted_iota(jnp.int32, (c, b), 0)
    hit = cls == lab_ref[...]
    ce_ref[0, 0] = -jnp.sum(jnp.where(hit, logp, 0.0)) / b


def _stage2_body(gt_ref, y_ref, aux_ref, ce_ref, res_ref):
    c, b = y_ref.shape
    g = gt_ref[0:c, :]
    y = y_ref[...]
    dot = BETA * jnp.sum(g * y, axis=0, keepdims=True) + aux_ref[0:1, :]
    elr = jnp.log(1.0 - dot)
    res_ref[0, 0] = ce_ref[0, 0] + LAMBDA_ * (jnp.sum(elr) / b)


def kernel(index, output, label, target):
    batch, ncls = output.shape
    idx = index.astype(jnp.int32)
    y_t, aux, ce = pl.pallas_call(
        _stage1_body,
        out_shape=(
            jax.ShapeDtypeStruct((ncls, batch), jnp.float32),
            jax.ShapeDtypeStruct((8, batch), jnp.float32),
            jax.ShapeDtypeStruct((1, 1), jnp.float32),
        ),
        in_specs=[
            pl.BlockSpec(memory_space=pltpu.VMEM),
            pl.BlockSpec(memory_space=pltpu.VMEM),
        ],
        out_specs=(
            pl.BlockSpec(memory_space=pltpu.VMEM),
            pl.BlockSpec(memory_space=pltpu.VMEM),
            pl.BlockSpec(memory_space=pltpu.SMEM),
        ),
    )(output.T, label.astype(jnp.int32).reshape(1, batch))
    gathered_t = _make_sc_gather(batch, ncls)(idx, target.T)
    res = pl.pallas_call(
        _stage2_body,
        out_shape=jax.ShapeDtypeStruct((1, 1), jnp.float32),
        in_specs=[
            pl.BlockSpec(memory_space=pltpu.VMEM),
            pl.BlockSpec(memory_space=pltpu.VMEM),
            pl.BlockSpec(memory_space=pltpu.VMEM),
            pl.BlockSpec(memory_space=pltpu.SMEM),
        ],
        out_specs=pl.BlockSpec(memory_space=pltpu.SMEM),
    )(gathered_t, y_t, aux, ce)
    return res[0, 0]
